# baseline (speedup 1.0000x reference)
import functools

import jax
import jax.numpy as jnp
from jax import lax
from jax.experimental import pallas as pl
from jax.experimental.pallas import tpu as pltpu
from jax.experimental.pallas import tpu_sc as plsc

_C = 100
_CP = 128
_D = 256
_B = 4096
_TLAMBDA = 0.01

_NC = 2
_NS = 16
_NW = _NC * _NS
_BPW = _B // _NW
_LANES = 16


def _r_tc_kernel(cp_ref, r_ref):
    cp = cp_ref[...]
    g = lax.dot_general(cp, cp, (((1,), (1,)), ((), ())),
                        preferred_element_type=jnp.float32)
    norm2 = jnp.sum(cp * cp, axis=1)
    n = jnp.sqrt(norm2)
    n_safe = jnp.where(n > 0.0, n, 1.0)
    cos = g / (n_safe[:, None] * n_safe[None, :])
    row = lax.broadcasted_iota(jnp.int32, (_CP, _CP), 0)
    col = lax.broadcasted_iota(jnp.int32, (_CP, _CP), 1)
    s_valid = jnp.sum(jnp.where(col < _C, cos, 0.0), axis=1)
    diag = jnp.sum(jnp.where(row == col, cos, 0.0), axis=1)
    r = s_valid - diag + (_C - 1.0)
    r_ref[...] = r.reshape(1, _CP)


_sc_mesh = plsc.VectorSubcoreMesh(core_axis_name="c", subcore_axis_name="s")

_DA = _D + _LANES


@functools.partial(
    pl.kernel,
    mesh=_sc_mesh,
    compiler_params=pltpu.CompilerParams(use_tc_tiling_on_sc=False),
    out_type=jax.ShapeDtypeStruct((_NW, _LANES), jnp.float32),
    scratch_types=[
        pltpu.VMEM((_BPW,), jnp.int32),
        pltpu.VMEM((_BPW, _DA), jnp.float32),
        pltpu.VMEM((_BPW, _D), jnp.float32),
        pltpu.VMEM((_LANES,), jnp.float32),
        pltpu.SemaphoreType.DMA,
    ],
)
def _sc_kernel(label_hbm, feat_hbm, aug_hbm, out_hbm,
               idx_v, gath_v, feat_v, out_v, sem):
    wid = lax.axis_index("s") * _NC + lax.axis_index("c")
    base = wid * _BPW
    pltpu.sync_copy(label_hbm.at[pl.ds(base, _BPW)], idx_v)
    cp_feat = pltpu.async_copy(feat_hbm.at[pl.ds(base, _BPW)], feat_v, sem)
    cp_gath = pltpu.async_copy(aug_hbm.at[idx_v], gath_v, sem)
    cp_feat.wait()
    cp_gath.wait()

    def row_body(i, carry):
        acc_c, acc_i = carry
        a = acc_c
        for k in range(_D // _LANES):
            dlt = (feat_v[i, pl.ds(k * _LANES, _LANES)]
                   - gath_v[i, pl.ds(k * _LANES, _LANES)])
            a = a + dlt * dlt
        b = acc_i + gath_v[i, pl.ds(_D, _LANES)]
        return (a, b)

    zero = jnp.zeros((_LANES,), jnp.float32)
    acc_c, acc_i = lax.fori_loop(0, _BPW, row_body, (zero, zero))

    out_v[...] = _TLAMBDA * acc_i + acc_c * (0.5 / _B)
    pltpu.sync_copy(out_v, out_hbm.at[wid])


def kernel(label, feat, centers, tlabel):
    del tlabel
    centers = centers.astype(jnp.float32)
    cp = jnp.pad(centers, ((0, _CP - _C), (0, 0)))
    r2d = pl.pallas_call(
        _r_tc_kernel,
        out_shape=jax.ShapeDtypeStruct((1, _CP), jnp.float32),
    )(cp)
    rpad = jnp.zeros((_C, _LANES), jnp.float32).at[:, 0].set(r2d[0, :_C])
    aug = jnp.concatenate([centers, rpad], axis=1)
    parts = _sc_kernel(label.astype(jnp.int32), feat.astype(jnp.float32), aug)
    return jnp.sum(parts)


# speedup vs baseline: 1.2451x; 1.2451x over previous
import functools

import jax
import jax.numpy as jnp
from jax import lax
from jax.experimental import pallas as pl
from jax.experimental.pallas import tpu as pltpu
from jax.experimental.pallas import tpu_sc as plsc

_C = 100
_CP = 128
_D = 256
_B = 4096
_TLAMBDA = 0.01

_NC = 2
_NS = 16
_NW = _NC * _NS
_BPW = _B // _NW
_LANES = 16

_sc_mesh = plsc.VectorSubcoreMesh(core_axis_name="c", subcore_axis_name="s")


@functools.partial(
    pl.kernel,
    mesh=_sc_mesh,
    out_type=jax.ShapeDtypeStruct((_NW, _LANES), jnp.float32),
    scratch_types=[
        pltpu.VMEM((_BPW,), jnp.int32),
        pltpu.VMEM((_BPW, _D), jnp.float32),
        pltpu.VMEM((_BPW, _D), jnp.float32),
        pltpu.VMEM((_LANES,), jnp.float32),
        pltpu.SemaphoreType.DMA,
    ],
)
def _sc_kernel(label_hbm, feat_hbm, centers_hbm, out_hbm,
               idx_v, gath_v, feat_v, out_v, sem):
    wid = lax.axis_index("s") * _NC + lax.axis_index("c")
    base = wid * _BPW
    pltpu.sync_copy(label_hbm.at[pl.ds(base, _BPW)], idx_v)
    cp_feat = pltpu.async_copy(feat_hbm.at[pl.ds(base, _BPW)], feat_v, sem)
    cp_gath = pltpu.async_copy(centers_hbm.at[idx_v], gath_v, sem)
    cp_feat.wait()
    cp_gath.wait()

    def row_body(i, acc):
        a = acc
        for k in range(_D // _LANES):
            dlt = (feat_v[i, pl.ds(k * _LANES, _LANES)]
                   - gath_v[i, pl.ds(k * _LANES, _LANES)])
            a = a + dlt * dlt
        return a

    acc_c = lax.fori_loop(0, _BPW, row_body, jnp.zeros((_LANES,), jnp.float32))
    out_v[...] = acc_c
    pltpu.sync_copy(out_v, out_hbm.at[wid])


def _tc_combine_kernel(cen_ref, lab_ref, part_ref, out_ref):
    cen = cen_ref[...]
    g = lax.dot_general(cen, cen, (((1,), (1,)), ((), ())),
                        preferred_element_type=jnp.float32)
    norm2 = jnp.sum(cen * cen, axis=1)
    n = jnp.sqrt(norm2)
    n_safe = jnp.where(n > 0.0, n, 1.0)
    cos = g / (n_safe[:, None] * n_safe[None, :])
    row = lax.broadcasted_iota(jnp.int32, (_C, _C), 0)
    col = lax.broadcasted_iota(jnp.int32, (_C, _C), 1)
    diag = jnp.sum(jnp.where(row == col, cos, 0.0), axis=1)
    r = jnp.sum(cos, axis=1) - diag + (_C - 1.0)

    lab = lab_ref[...]
    cls = lax.broadcasted_iota(jnp.int32, (_B // _CP, _CP, _C), 2)
    onehot = (lab[:, :, None] == cls).astype(jnp.float32)
    count = jnp.sum(onehot, axis=(0, 1))
    island = jnp.sum(count * r)

    center_sum = jnp.sum(part_ref[...])
    total = _TLAMBDA * island + center_sum * (0.5 / _B)
    out_ref[...] = jnp.full((1, 1), total, jnp.float32)


def kernel(label, feat, centers, tlabel):
    del tlabel
    label = label.astype(jnp.int32)
    centers = centers.astype(jnp.float32)
    parts = _sc_kernel(label, feat.astype(jnp.float32), centers)
    total = pl.pallas_call(
        _tc_combine_kernel,
        out_shape=jax.ShapeDtypeStruct((1, 1), jnp.float32),
    )(centers, label.reshape(_B // _CP, _CP), parts)
    return total.reshape(())


# speedup vs baseline: 1.2848x; 1.0319x over previous
import functools

import jax
import jax.numpy as jnp
from jax import lax
from jax.experimental import pallas as pl
from jax.experimental.pallas import tpu as pltpu
from jax.experimental.pallas import tpu_sc as plsc

_C = 100
_D = 256
_DH = 128
_B = 4096
_TLAMBDA = 0.01

_NC = 2
_NS = 16
_NW = _NC * _NS
_BPW = _B // _NW
_LANES = 16

_sc_mesh = plsc.VectorSubcoreMesh(core_axis_name="c", subcore_axis_name="s")


@functools.partial(
    pl.kernel,
    mesh=_sc_mesh,
    out_type=jax.ShapeDtypeStruct((_NW, _LANES), jnp.float32),
    scratch_types=[
        pltpu.VMEM((_BPW,), jnp.int32),
        pltpu.VMEM((_BPW,), jnp.int32),
        pltpu.VMEM((_BPW,), jnp.int32),
        pltpu.VMEM((_BPW, _DH), jnp.float32),
        pltpu.VMEM((_BPW, _DH), jnp.float32),
        pltpu.VMEM((_BPW, _D), jnp.float32),
        pltpu.VMEM((_LANES,), jnp.float32),
        pltpu.SemaphoreType.DMA,
        pltpu.SemaphoreType.DMA,
    ],
)
def _sc_kernel(label_hbm, feat_hbm, tab_hbm, out_hbm,
               idx_v, idxa_v, idxb_v, gath_a, gath_b, feat_v, out_v,
               sem, sem2):
    wid = lax.axis_index("s") * _NC + lax.axis_index("c")
    base = wid * _BPW
    pltpu.sync_copy(label_hbm.at[pl.ds(base, _BPW)], idx_v)
    cp_feat = pltpu.async_copy(feat_hbm.at[pl.ds(base, _BPW)], feat_v, sem)

    def mkidx(j, _):
        two = idx_v[pl.ds(j * _LANES, _LANES)] * 2
        idxa_v[pl.ds(j * _LANES, _LANES)] = two
        idxb_v[pl.ds(j * _LANES, _LANES)] = two + 1
        return 0
    lax.fori_loop(0, _BPW // _LANES, mkidx, 0)

    cp_ga = pltpu.async_copy(tab_hbm.at[idxa_v], gath_a, sem2)
    cp_gb = pltpu.async_copy(tab_hbm.at[idxb_v], gath_b, sem2)
    cp_feat.wait()
    cp_ga.wait()
    cp_gb.wait()

    def row_body(i, acc):
        a = acc
        for k in range(_DH // _LANES):
            dlt = (feat_v[i, pl.ds(k * _LANES, _LANES)]
                   - gath_a[i, pl.ds(k * _LANES, _LANES)])
            a = a + dlt * dlt
        for k in range(_DH // _LANES):
            dlt = (feat_v[i, pl.ds(_DH + k * _LANES, _LANES)]
                   - gath_b[i, pl.ds(k * _LANES, _LANES)])
            a = a + dlt * dlt
        return a

    acc_c = lax.fori_loop(0, _BPW, row_body, jnp.zeros((_LANES,), jnp.float32))
    out_v[...] = acc_c
    pltpu.sync_copy(out_v, out_hbm.at[wid])


def _tc_combine_kernel(cen_ref, lab_ref, part_ref, out_ref):
    cen = cen_ref[...]
    g = lax.dot_general(cen, cen, (((1,), (1,)), ((), ())),
                        preferred_element_type=jnp.float32)
    norm2 = jnp.sum(cen * cen, axis=1)
    n = jnp.sqrt(norm2)
    n_safe = jnp.where(n > 0.0, n, 1.0)
    cos = g / (n_safe[:, None] * n_safe[None, :])
    row = lax.broadcasted_iota(jnp.int32, (_C, _C), 0)
    col = lax.broadcasted_iota(jnp.int32, (_C, _C), 1)
    diag = jnp.sum(jnp.where(row == col, cos, 0.0), axis=1)
    r = jnp.sum(cos, axis=1) - diag + (_C - 1.0)

    lab = lab_ref[...]
    cls = lax.broadcasted_iota(jnp.int32, (_B // 128, 128, _C), 2)
    onehot = (lab[:, :, None] == cls).astype(jnp.float32)
    count = jnp.sum(onehot, axis=(0, 1))
    island = jnp.sum(count * r)

    center_sum = jnp.sum(part_ref[...])
    total = _TLAMBDA * island + center_sum * (0.5 / _B)
    out_ref[...] = jnp.full((1, 1), total, jnp.float32)


def kernel(label, feat, centers, tlabel):
    del tlabel
    label = label.astype(jnp.int32)
    centers = centers.astype(jnp.float32)
    tab = centers.reshape(2 * _C, _DH)
    parts = _sc_kernel(label, feat.astype(jnp.float32), tab)
    total = pl.pallas_call(
        _tc_combine_kernel,
        out_shape=jax.ShapeDtypeStruct((1, 1), jnp.float32),
    )(centers, label.reshape(_B // 128, 128), parts)
    return total.reshape(())


# speedup vs baseline: 1.3013x; 1.0128x over previous
import functools

import jax
import jax.numpy as jnp
from jax import lax
from jax.experimental import pallas as pl
from jax.experimental.pallas import tpu as pltpu
from jax.experimental.pallas import tpu_sc as plsc

_C = 100
_D = 256
_DH = 128
_B = 4096
_TLAMBDA = 0.01

_NC = 2
_NS = 16
_NW = _NC * _NS
_BPW = _B // _NW
_LANES = 16
_NCHUNK = 4
_CH = _BPW // _NCHUNK

_sc_mesh = plsc.VectorSubcoreMesh(core_axis_name="c", subcore_axis_name="s")


@functools.partial(
    pl.kernel,
    mesh=_sc_mesh,
    out_type=jax.ShapeDtypeStruct((_NW, _LANES), jnp.float32),
    scratch_types=[
        pltpu.VMEM((_BPW,), jnp.int32),
        pltpu.VMEM((_BPW,), jnp.int32),
        pltpu.VMEM((_BPW,), jnp.int32),
        pltpu.VMEM((_BPW, _DH), jnp.float32),
        pltpu.VMEM((_BPW, _DH), jnp.float32),
        pltpu.VMEM((_BPW, _D), jnp.float32),
        pltpu.VMEM((_LANES,), jnp.float32),
        pltpu.SemaphoreType.DMA,
        pltpu.SemaphoreType.DMA,
        pltpu.SemaphoreType.DMA,
    ],
)
def _sc_kernel(label_hbm, feat_hbm, tab_hbm, out_hbm,
               idx_v, idxa_v, idxb_v, gath_a, gath_b, feat_v, out_v,
               semf, sema, semb):
    wid = lax.axis_index("s") * _NC + lax.axis_index("c")
    base = wid * _BPW
    pltpu.sync_copy(label_hbm.at[pl.ds(base, _BPW)], idx_v)

    def mkidx(j, _):
        two = idx_v[pl.ds(j * _LANES, _LANES)] * 2
        idxa_v[pl.ds(j * _LANES, _LANES)] = two
        idxb_v[pl.ds(j * _LANES, _LANES)] = two + 1
        return 0
    lax.fori_loop(0, _BPW // _LANES, mkidx, 0)

    copies = []
    for c in range(_NCHUNK):
        s = c * _CH
        copies.append((
            pltpu.async_copy(feat_hbm.at[pl.ds(base + s, _CH)],
                             feat_v.at[pl.ds(s, _CH)], semf),
            pltpu.async_copy(tab_hbm.at[idxa_v.at[pl.ds(s, _CH)]],
                             gath_a.at[pl.ds(s, _CH)], sema),
            pltpu.async_copy(tab_hbm.at[idxb_v.at[pl.ds(s, _CH)]],
                             gath_b.at[pl.ds(s, _CH)], semb),
        ))

    def row_body(i, acc):
        a = acc
        for k in range(_DH // _LANES):
            dlt = (feat_v[i, pl.ds(k * _LANES, _LANES)]
                   - gath_a[i, pl.ds(k * _LANES, _LANES)])
            a = a + dlt * dlt
        for k in range(_DH // _LANES):
            dlt = (feat_v[i, pl.ds(_DH + k * _LANES, _LANES)]
                   - gath_b[i, pl.ds(k * _LANES, _LANES)])
            a = a + dlt * dlt
        return a

    acc_c = jnp.zeros((_LANES,), jnp.float32)
    for c in range(_NCHUNK):
        for cp in copies[c]:
            cp.wait()
        acc_c = lax.fori_loop(c * _CH, (c + 1) * _CH, row_body, acc_c)

    out_v[...] = acc_c
    pltpu.sync_copy(out_v, out_hbm.at[wid])


def _tc_island_kernel(cen_ref, lab_ref, out_ref):
    cen = cen_ref[...]
    g = lax.dot_general(cen, cen, (((1,), (1,)), ((), ())),
                        preferred_element_type=jnp.float32)
    norm2 = jnp.sum(cen * cen, axis=1)
    n = jnp.sqrt(norm2)
    n_safe = jnp.where(n > 0.0, n, 1.0)
    cos = g / (n_safe[:, None] * n_safe[None, :])
    row = lax.broadcasted_iota(jnp.int32, (_C, _C), 0)
    col = lax.broadcasted_iota(jnp.int32, (_C, _C), 1)
    diag = jnp.sum(jnp.where(row == col, cos, 0.0), axis=1)
    r = jnp.sum(cos, axis=1) - diag + (_C - 1.0)

    lab = lab_ref[...]
    cls = lax.broadcasted_iota(jnp.int32, (_B // 128, 128, _C), 2)
    onehot = (lab[:, :, None] == cls).astype(jnp.float32)
    count = jnp.sum(onehot, axis=(0, 1))
    island = jnp.sum(count * r)
    out_ref[...] = jnp.full((1, 1), island, jnp.float32)


def kernel(label, feat, centers, tlabel):
    del tlabel
    label = label.astype(jnp.int32)
    centers = centers.astype(jnp.float32)
    tab = centers.reshape(2 * _C, _DH)
    parts = _sc_kernel(label, feat.astype(jnp.float32), tab)
    island = pl.pallas_call(
        _tc_island_kernel,
        out_shape=jax.ShapeDtypeStruct((1, 1), jnp.float32),
    )(centers, label.reshape(_B // 128, 128))
    return _TLAMBDA * island.reshape(()) + jnp.sum(parts) * (0.5 / _B)


# speedup vs baseline: 1.3648x; 1.0488x over previous
import functools

import jax
import jax.numpy as jnp
from jax import lax
from jax.experimental import pallas as pl
from jax.experimental.pallas import tpu as pltpu
from jax.experimental.pallas import tpu_sc as plsc

_C = 100
_D = 256
_DH = 128
_B = 4096
_BSC = 2048
_TLAMBDA = 0.01

_NC = 2
_NS = 16
_NW = _NC * _NS
_BPW = _BSC // _NW
_LANES = 16
_NCHUNK = 2
_CH = _BPW // _NCHUNK

_sc_mesh = plsc.VectorSubcoreMesh(core_axis_name="c", subcore_axis_name="s")


@functools.partial(
    pl.kernel,
    mesh=_sc_mesh,
    out_type=jax.ShapeDtypeStruct((_NW, _LANES), jnp.float32),
    scratch_types=[
        pltpu.VMEM((_BPW,), jnp.int32),
        pltpu.VMEM((_BPW,), jnp.int32),
        pltpu.VMEM((_BPW,), jnp.int32),
        pltpu.VMEM((_BPW, _DH), jnp.float32),
        pltpu.VMEM((_BPW, _DH), jnp.float32),
        pltpu.VMEM((_BPW, _D), jnp.float32),
        pltpu.VMEM((_LANES,), jnp.float32),
        pltpu.SemaphoreType.DMA,
        pltpu.SemaphoreType.DMA,
        pltpu.SemaphoreType.DMA,
    ],
)
def _sc_kernel(label_hbm, feat_hbm, tab_hbm, out_hbm,
               idx_v, idxa_v, idxb_v, gath_a, gath_b, feat_v, out_v,
               semf, sema, semb):
    wid = lax.axis_index("s") * _NC + lax.axis_index("c")
    base = wid * _BPW
    pltpu.sync_copy(label_hbm.at[pl.ds(base, _BPW)], idx_v)

    def mkidx(j, _):
        two = idx_v[pl.ds(j * _LANES, _LANES)] * 2
        idxa_v[pl.ds(j * _LANES, _LANES)] = two
        idxb_v[pl.ds(j * _LANES, _LANES)] = two + 1
        return 0
    lax.fori_loop(0, _BPW // _LANES, mkidx, 0)

    copies = []
    for c in range(_NCHUNK):
        s = c * _CH
        copies.append((
            pltpu.async_copy(feat_hbm.at[pl.ds(base + s, _CH)],
                             feat_v.at[pl.ds(s, _CH)], semf),
            pltpu.async_copy(tab_hbm.at[idxa_v.at[pl.ds(s, _CH)]],
                             gath_a.at[pl.ds(s, _CH)], sema),
            pltpu.async_copy(tab_hbm.at[idxb_v.at[pl.ds(s, _CH)]],
                             gath_b.at[pl.ds(s, _CH)], semb),
        ))

    def row_body(i, acc):
        a = acc
        for k in range(_DH // _LANES):
            dlt = (feat_v[i, pl.ds(k * _LANES, _LANES)]
                   - gath_a[i, pl.ds(k * _LANES, _LANES)])
            a = a + dlt * dlt
        for k in range(_DH // _LANES):
            dlt = (feat_v[i, pl.ds(_DH + k * _LANES, _LANES)]
                   - gath_b[i, pl.ds(k * _LANES, _LANES)])
            a = a + dlt * dlt
        return a

    acc_c = jnp.zeros((_LANES,), jnp.float32)
    for c in range(_NCHUNK):
        for cp in copies[c]:
            cp.wait()
        acc_c = lax.fori_loop(c * _CH, (c + 1) * _CH, row_body, acc_c)

    out_v[...] = acc_c
    pltpu.sync_copy(out_v, out_hbm.at[wid])


def _tc_kernel(cen_ref, lab_ref, feat_ref, out_ref):
    cen = cen_ref[...]
    g = lax.dot_general(cen, cen, (((1,), (1,)), ((), ())),
                        preferred_element_type=jnp.float32)
    norm2 = jnp.sum(cen * cen, axis=1)
    n = jnp.sqrt(norm2)
    n_safe = jnp.where(n > 0.0, n, 1.0)
    cos = g / (n_safe[:, None] * n_safe[None, :])
    row = lax.broadcasted_iota(jnp.int32, (_C, _C), 0)
    col = lax.broadcasted_iota(jnp.int32, (_C, _C), 1)
    diag = jnp.sum(jnp.where(row == col, cos, 0.0), axis=1)
    r = jnp.sum(cos, axis=1) - diag + (_C - 1.0)

    labc = lab_ref[...]
    cls = lax.broadcasted_iota(jnp.int32, (_B, _C), 1)
    ohall = (labc == cls).astype(jnp.float32)
    count = jnp.sum(ohall, axis=0)
    island = jnp.sum(count * r)

    oh_hi = ohall[_BSC:, :]
    cb = lax.dot_general(oh_hi, cen, (((1,), (0,)), ((), ())),
                         preferred_element_type=jnp.float32)
    fh = feat_ref[...]
    dlt = fh - cb
    center_hi = jnp.sum(dlt * dlt)

    total = _TLAMBDA * island + center_hi * (0.5 / _B)
    out_ref[...] = jnp.full((1, 1), total, jnp.float32)


def kernel(label, feat, centers, tlabel):
    del tlabel
    label = label.astype(jnp.int32)
    feat = feat.astype(jnp.float32)
    centers = centers.astype(jnp.float32)
    tab = centers.reshape(2 * _C, _DH)
    parts = _sc_kernel(label, feat, tab)
    tc_total = pl.pallas_call(
        _tc_kernel,
        grid=(1,),
        in_specs=[
            pl.BlockSpec((_C, _D), lambda i: (0, 0)),
            pl.BlockSpec((_B, 1), lambda i: (0, 0)),
            pl.BlockSpec((_BSC, _D), lambda i: (1, 0)),
        ],
        out_specs=pl.BlockSpec((1, 1), lambda i: (0, 0)),
        out_shape=jax.ShapeDtypeStruct((1, 1), jnp.float32),
    )(centers, label[:, None], feat)
    return tc_total.reshape(()) + jnp.sum(parts) * (0.5 / _B)


# speedup vs baseline: 1.3886x; 1.0174x over previous
import functools

import jax
import jax.numpy as jnp
from jax import lax
from jax.experimental import pallas as pl
from jax.experimental.pallas import tpu as pltpu
from jax.experimental.pallas import tpu_sc as plsc

_C = 100
_D = 256
_DH = 128
_B = 4096
_BSC = 2048
_TLAMBDA = 0.01

_NC = 2
_NS = 16
_NW = _NC * _NS
_BPW = _BSC // _NW
_LANES = 16
_NCHUNK = 4
_CH = _BPW // _NCHUNK

_sc_mesh = plsc.VectorSubcoreMesh(core_axis_name="c", subcore_axis_name="s")


@functools.partial(
    pl.kernel,
    mesh=_sc_mesh,
    out_type=jax.ShapeDtypeStruct((_NW, _LANES), jnp.float32),
    scratch_types=[
        pltpu.VMEM((_BPW,), jnp.int32),
        pltpu.VMEM((_BPW,), jnp.int32),
        pltpu.VMEM((_BPW,), jnp.int32),
        pltpu.VMEM((_BPW, _DH), jnp.float32),
        pltpu.VMEM((_BPW, _DH), jnp.float32),
        pltpu.VMEM((_BPW, _D), jnp.float32),
        pltpu.VMEM((_LANES,), jnp.float32),
        pltpu.SemaphoreType.DMA,
        pltpu.SemaphoreType.DMA,
        pltpu.SemaphoreType.DMA,
    ],
)
def _sc_kernel(label_hbm, feat_hbm, tab_hbm, out_hbm,
               idx_v, idxa_v, idxb_v, gath_a, gath_b, feat_v, out_v,
               semf, sema, semb):
    wid = lax.axis_index("s") * _NC + lax.axis_index("c")
    base = wid * _BPW
    pltpu.sync_copy(label_hbm.at[pl.ds(base, _BPW)], idx_v)

    def mkidx(j, _):
        two = idx_v[pl.ds(j * _LANES, _LANES)] * 2
        idxa_v[pl.ds(j * _LANES, _LANES)] = two
        idxb_v[pl.ds(j * _LANES, _LANES)] = two + 1
        return 0
    lax.fori_loop(0, _BPW // _LANES, mkidx, 0)

    copies = []
    for c in range(_NCHUNK):
        s = c * _CH
        copies.append((
            pltpu.async_copy(feat_hbm.at[pl.ds(base + s, _CH)],
                             feat_v.at[pl.ds(s, _CH)], semf),
            pltpu.async_copy(tab_hbm.at[idxa_v.at[pl.ds(s, _CH)]],
                             gath_a.at[pl.ds(s, _CH)], sema),
            pltpu.async_copy(tab_hbm.at[idxb_v.at[pl.ds(s, _CH)]],
                             gath_b.at[pl.ds(s, _CH)], semb),
        ))

    def row_body(i, acc):
        a = acc
        for k in range(_DH // _LANES):
            dlt = (feat_v[i, pl.ds(k * _LANES, _LANES)]
                   - gath_a[i, pl.ds(k * _LANES, _LANES)])
            a = a + dlt * dlt
        for k in range(_DH // _LANES):
            dlt = (feat_v[i, pl.ds(_DH + k * _LANES, _LANES)]
                   - gath_b[i, pl.ds(k * _LANES, _LANES)])
            a = a + dlt * dlt
        return a

    acc_c = jnp.zeros((_LANES,), jnp.float32)
    for c in range(_NCHUNK):
        for cp in copies[c]:
            cp.wait()
        acc_c = lax.fori_loop(c * _CH, (c + 1) * _CH, row_body, acc_c)

    out_v[...] = acc_c
    pltpu.sync_copy(out_v, out_hbm.at[wid])


def _tc_kernel(cen_ref, lab_ref, feat_ref, out_ref):
    cen = cen_ref[...]
    g = lax.dot_general(cen, cen, (((1,), (1,)), ((), ())),
                        preferred_element_type=jnp.float32)
    norm2 = jnp.sum(cen * cen, axis=1)
    n = jnp.sqrt(norm2)
    n_safe = jnp.where(n > 0.0, n, 1.0)
    cos = g / (n_safe[:, None] * n_safe[None, :])
    row = lax.broadcasted_iota(jnp.int32, (_C, _C), 0)
    col = lax.broadcasted_iota(jnp.int32, (_C, _C), 1)
    diag = jnp.sum(jnp.where(row == col, cos, 0.0), axis=1)
    r = jnp.sum(cos, axis=1) - diag + (_C - 1.0)

    labc = lab_ref[...]
    cls = lax.broadcasted_iota(jnp.int32, (_B, _C), 1)
    ohall = (labc == cls).astype(jnp.float32)
    count = jnp.sum(ohall, axis=0)
    island = jnp.sum(count * r)

    oh_hi = ohall[_BSC:, :]
    cb = lax.dot_general(oh_hi, cen, (((1,), (0,)), ((), ())),
                         preferred_element_type=jnp.float32)
    fh = feat_ref[...]
    dlt = fh - cb
    center_hi = jnp.sum(dlt * dlt)

    total = _TLAMBDA * island + center_hi * (0.5 / _B)
    out_ref[...] = jnp.full((1, 1), total, jnp.float32)


def kernel(label, feat, centers, tlabel):
    del tlabel
    label = label.astype(jnp.int32)
    feat = feat.astype(jnp.float32)
    centers = centers.astype(jnp.float32)
    tab = centers.reshape(2 * _C, _DH)
    parts = _sc_kernel(label, feat, tab)
    tc_total = pl.pallas_call(
        _tc_kernel,
        grid=(1,),
        in_specs=[
            pl.BlockSpec((_C, _D), lambda i: (0, 0)),
            pl.BlockSpec((_B, 1), lambda i: (0, 0)),
            pl.BlockSpec((_BSC, _D), lambda i: (1, 0)),
        ],
        out_specs=pl.BlockSpec((1, 1), lambda i: (0, 0)),
        out_shape=jax.ShapeDtypeStruct((1, 1), jnp.float32),
    )(centers, label[:, None], feat)
    return tc_total.reshape(()) + jnp.sum(parts) * (0.5 / _B)
